# baseline (speedup 1.0000x reference)
import jax
import jax.numpy as jnp
import numpy as np
from jax.experimental import pallas as pl

NUM_LAYERS = 3
INNER_STEPS = 4
INNER_LR = 0.1
TRANS_DIST = 1.0


def _mpnn(h0, edge_index, efeat, Wproj, We1, We2, Wself, Wnbr):
    h = jax.nn.relu(h0 @ Wproj)
    src = edge_index[0]
    dst = edge_index[1]
    gate = jax.nn.relu(efeat @ We1) @ We2
    for _ in range(NUM_LAYERS):
        msg = gate * h[src]
        agg = jax.ops.segment_sum(msg, dst, num_segments=h.shape[0])
        h = jax.nn.relu(h @ Wself + agg @ Wnbr)
    return h


def kernel(rec_x, rec_coord, rec_edge_index, rec_edge_attr, lig_x, lig_coord, lig_edge_index, lig_edge_attr, Wrn, Wln, Wre, Wle, rec_Wproj, rec_We1, rec_We2, rec_Wself, rec_Wnbr, lig_Wproj, lig_We1, lig_We2, lig_Wself, lig_Wnbr):
    rec_hid = rec_x @ Wrn
    lig_hid = lig_x @ Wln
    rec_e = rec_edge_attr @ Wre
    lig_e = lig_edge_attr @ Wle
    rec_feat = _mpnn(rec_hid, rec_edge_index, rec_e, rec_Wproj, rec_We1, rec_We2, rec_Wself, rec_Wnbr)
    lig_feat = _mpnn(lig_hid, lig_edge_index, lig_e, lig_Wproj, lig_We1, lig_We2, lig_Wself, lig_Wnbr)
    lig_c = lig_coord - lig_coord.mean(axis=0)
    rec_c = rec_coord - rec_coord.mean(axis=0)
    kr = jax.random.key(7)
    rot, _ = jnp.linalg.qr(jax.random.normal(jax.random.fold_in(kr, 0), (3, 3), dtype=jnp.float32))
    trans = jax.random.normal(jax.random.fold_in(kr, 1), (3,), dtype=jnp.float32) * TRANS_DIST

    def energy(rot_, trans_):
        new = lig_c @ rot_.T + trans_
        atn = lig_feat @ rec_feat.T
        d2 = jnp.sum(new * new, axis=1)[:, None] + jnp.sum(rec_c * rec_c, axis=1)[None, :] - 2.0 * (new @ rec_c.T)
        d = jnp.sqrt(jnp.maximum(d2, 1e-12))
        return (atn / (d ** 2)).mean()

    Us = []
    for _ in range(INNER_STEPS):
        U, (g_rot, g_trans) = jax.value_and_grad(energy, argnums=(0, 1))(rot, trans)
        Us.append(U)
        rot = rot - g_rot * INNER_LR
        trans = trans - g_trans * INNER_LR
    return jnp.stack(Us)


# speedup vs baseline: 2.6025x; 2.6025x over previous
import functools

import jax
import jax.numpy as jnp
from jax import lax
from jax.experimental import pallas as pl
from jax.experimental.pallas import tpu as pltpu
from jax.experimental.pallas import tpu_sc as plsc

_NUM_LAYERS = 3
_INNER_STEPS = 4
_INNER_LR = 0.1
_TRANS_DIST = 1.0

_NC = 2
_NS = 16
_NW = _NC * _NS


def _make_edge_agg(n_nodes, n_edges, K):
    assert n_edges % (_NW * K) == 0 and K % 8 == 0 and K <= 128
    chunk = n_edges // _NW
    iters = chunk // K
    slab = ((n_nodes + _NS - 1) // _NS + 7) // 8 * 8
    n_pad = slab * _NS
    ZB = 80 if slab % 80 == 0 else (64 if slab % 64 == 0 else 8)
    assert slab % ZB == 0
    mesh = plsc.VectorSubcoreMesh(core_axis_name="c", subcore_axis_name="s",
                                  num_cores=_NC, num_subcores=_NS)

    @functools.partial(
        pl.kernel,
        out_type=jax.ShapeDtypeStruct((_NC * n_pad, 128), jnp.float32),
        mesh=mesh,
        scratch_types=[
            pltpu.VMEM((K,), jnp.int32),
            pltpu.VMEM((K,), jnp.int32),
            pltpu.VMEM((K, 128), jnp.float32),
            pltpu.VMEM((K, 128), jnp.float32),
            pltpu.VMEM((ZB, 128), jnp.float32),
            pltpu.VMEM_SHARED((n_pad, 128), jnp.float32),
            pltpu.SemaphoreType.DMA,
        ],
    )
    def body(h, gate, src, dst, out, src_v, dst_v, rows_v, gate_v, zbuf, shared, sem):
        c = lax.axis_index("c")
        s = lax.axis_index("s")

        def zfill(i, carry):
            for r in range(8):
                zbuf[i, pl.ds(r * 16, 16)] = jnp.zeros((16,), jnp.float32)
            return carry

        lax.fori_loop(0, ZB, zfill, 0)
        off = s * slab

        def zslab(i, carry):
            pltpu.sync_copy(zbuf, shared.at[pl.ds(off + i * ZB, ZB)])
            return carry

        lax.fori_loop(0, slab // ZB, zslab, 0)
        plsc.subcore_barrier()

        base = (c * _NS + s) * chunk

        def eloop(i, carry):
            b = base + i * K
            pltpu.sync_copy(src.at[pl.ds(b, K)], src_v)
            pltpu.sync_copy(dst.at[pl.ds(b, K)], dst_v)
            cp = pltpu.async_copy(h.at[src_v], rows_v, sem)
            pltpu.sync_copy(gate.at[pl.ds(b, K)], gate_v)
            cp.wait()

            def mul(j, carry2):
                for r in range(8):
                    sl = pl.ds(r * 16, 16)
                    rows_v[j, sl] = rows_v[j, sl] * gate_v[j, sl]
                return carry2

            lax.fori_loop(0, K, mul, 0)
            pltpu.sync_copy(rows_v, shared.at[dst_v], add=True)
            return carry

        lax.fori_loop(0, iters, eloop, 0)
        plsc.subcore_barrier()
        pltpu.sync_copy(shared.at[pl.ds(off, slab)],
                        out.at[pl.ds(c * n_pad + off, slab)])

    return body, n_pad


def _node_embed(x, Wa, Wb, tile):
    n = x.shape[0]
    assert n % tile == 0

    def f(x_ref, wa_ref, wb_ref, o_ref):
        t = jnp.dot(x_ref[...].astype(jnp.bfloat16),
                    wa_ref[...].astype(jnp.bfloat16),
                    preferred_element_type=jnp.float32)
        o_ref[...] = jax.nn.relu(
            jnp.dot(t.astype(jnp.bfloat16), wb_ref[...].astype(jnp.bfloat16),
                    preferred_element_type=jnp.float32))

    return pl.pallas_call(
        f,
        grid=(n // tile,),
        in_specs=[
            pl.BlockSpec((tile, 128), lambda i: (i, 0)),
            pl.BlockSpec((128, 128), lambda i: (0, 0)),
            pl.BlockSpec((128, 128), lambda i: (0, 0)),
        ],
        out_specs=pl.BlockSpec((tile, 128), lambda i: (i, 0)),
        out_shape=jax.ShapeDtypeStruct((n, 128), jnp.float32),
    )(x, Wa, Wb)


def _edge_gate(eattr, We, We1, We2, tile):
    e, din = eattr.shape
    assert e % tile == 0
    deh = We1.shape[1]

    def f(e_ref, w_ref, w1_ref, w2_ref, o_ref):
        t = jnp.dot(e_ref[...].astype(jnp.bfloat16),
                    w_ref[...].astype(jnp.bfloat16),
                    preferred_element_type=jnp.float32)
        t = jnp.dot(t.astype(jnp.bfloat16), w1_ref[...].astype(jnp.bfloat16),
                    preferred_element_type=jnp.float32)
        o_ref[...] = jnp.dot(jax.nn.relu(t).astype(jnp.bfloat16),
                             w2_ref[...].astype(jnp.bfloat16),
                             preferred_element_type=jnp.float32)

    return pl.pallas_call(
        f,
        grid=(e // tile,),
        in_specs=[
            pl.BlockSpec((tile, din), lambda i: (i, 0)),
            pl.BlockSpec(We.shape, lambda i: (0, 0)),
            pl.BlockSpec(We1.shape, lambda i: (0, 0)),
            pl.BlockSpec(We2.shape, lambda i: (0, 0)),
        ],
        out_specs=pl.BlockSpec((tile, 128), lambda i: (i, 0)),
        out_shape=jax.ShapeDtypeStruct((e, 128), jnp.float32),
    )(eattr, We, We1, We2)


def _layer_update(h, a0, a1, Wself, Wnbr, tile):
    n = h.shape[0]
    assert n % tile == 0

    def f(h_ref, a0_ref, a1_ref, ws_ref, wn_ref, o_ref):
        agg = a0_ref[...] + a1_ref[...]
        o_ref[...] = jax.nn.relu(
            jnp.dot(h_ref[...].astype(jnp.bfloat16),
                    ws_ref[...].astype(jnp.bfloat16),
                    preferred_element_type=jnp.float32)
            + jnp.dot(agg.astype(jnp.bfloat16),
                      wn_ref[...].astype(jnp.bfloat16),
                      preferred_element_type=jnp.float32))

    return pl.pallas_call(
        f,
        grid=(n // tile,),
        in_specs=[
            pl.BlockSpec((tile, 128), lambda i: (i, 0)),
            pl.BlockSpec((tile, 128), lambda i: (i, 0)),
            pl.BlockSpec((tile, 128), lambda i: (i, 0)),
            pl.BlockSpec((128, 128), lambda i: (0, 0)),
            pl.BlockSpec((128, 128), lambda i: (0, 0)),
        ],
        out_specs=pl.BlockSpec((tile, 128), lambda i: (i, 0)),
        out_shape=jax.ShapeDtypeStruct((n, 128), jnp.float32),
    )(h, a0, a1, Wself, Wnbr)


def _dock(lig_feat, rec_feat_pad, lig_cpad, rec_crd_pad, rc_fat, R0, t0,
          nl, nr, TR):
    nrp = rec_feat_pad.shape[0]
    NT = nrp // TR
    c1 = -2.0 / (nl * nr)
    eps = 1e-12
    bf16 = jnp.bfloat16

    def f(lf_ref, rf_ref, lc_ref, rc_ref, rcf_ref, r0_ref, t0_ref, us_ref,
          atn_s, R_s, t_s, new_s, ln2_s, acc_s, wr_s, u_s):
        k = pl.program_id(0)
        j = pl.program_id(1)

        @pl.when(jnp.logical_and(k == 0, j == 0))
        def _init():
            R_s[...] = r0_ref[...]
            t_s[...] = t0_ref[...]

        @pl.when(k == 0)
        def _atn():
            atn_s[j] = lax.dot_general(
                lf_ref[...].astype(bf16), rf_ref[...].astype(bf16),
                (((1,), (1,)), ((), ())), preferred_element_type=jnp.float32)

        @pl.when(j == 0)
        def _step_init():
            new = jnp.dot(lc_ref[...].astype(bf16), R_s[...].astype(bf16),
                          preferred_element_type=jnp.float32) + t_s[0:1, :]
            new_s[...] = new
            nx = new[:, 0:1]
            ny = new[:, 1:2]
            nz = new[:, 2:3]
            ln2_s[...] = (nx * nx + nz * nz) + ny * ny
            acc_s[...] = jnp.zeros_like(acc_s)
            wr_s[...] = jnp.zeros_like(wr_s)
            u_s[...] = jnp.zeros_like(u_s)

        cross = lax.dot_general(new_s[...].astype(bf16), rcf_ref[...],
                                (((1,), (0,)), ((), ())),
                                preferred_element_type=jnp.float32)
        rx = rc_ref[0:1, :]
        ry = rc_ref[1:2, :]
        rz = rc_ref[2:3, :]
        rn2 = (rx * rx + rz * rz) + ry * ry
        d2 = ln2_s[...] + rn2 - 2.0 * cross
        m = jnp.maximum(d2, eps)
        dd = jnp.sqrt(m)
        dd2 = dd * dd
        atn = atn_s[j]
        q = atn / dd2
        u_s[...] += jnp.full((1, 128), jnp.sum(q), jnp.float32)
        w = q / dd2 * (d2 > eps).astype(jnp.float32)
        acc_s[:, 0:1] += jnp.sum(w, axis=1, keepdims=True)
        wr_s[:, 0:1] += jnp.sum(w * rx, axis=1, keepdims=True)
        wr_s[:, 1:2] += jnp.sum(w * ry, axis=1, keepdims=True)
        wr_s[:, 2:3] += jnp.sum(w * rz, axis=1, keepdims=True)

        @pl.when(j == NT - 1)
        def _finish():
            us_ref[pl.ds(k, 1), :] = u_s[...] * (1.0 / (nl * nr))
            G = c1 * (new_s[...] * acc_s[:, 0:1] - wr_s[...])
            gR = lax.dot_general(lc_ref[...].astype(bf16), G.astype(bf16),
                                 (((0,), (0,)), ((), ())),
                                 preferred_element_type=jnp.float32)
            gt = jnp.sum(G, axis=0, keepdims=True)
            R_s[...] = R_s[...] - _INNER_LR * gR
            t_s[0:1, :] = t_s[0:1, :] - _INNER_LR * gt

    return pl.pallas_call(
        f,
        grid=(_INNER_STEPS, NT),
        in_specs=[
            pl.BlockSpec((nl, 128), lambda k, j: (0, 0)),
            pl.BlockSpec((TR, 128), lambda k, j: (j, 0)),
            pl.BlockSpec((nl, 128), lambda k, j: (0, 0)),
            pl.BlockSpec((8, TR), lambda k, j: (0, j)),
            pl.BlockSpec((128, TR), lambda k, j: (0, j)),
            pl.BlockSpec((128, 128), lambda k, j: (0, 0)),
            pl.BlockSpec((8, 128), lambda k, j: (0, 0)),
        ],
        out_specs=pl.BlockSpec((_INNER_STEPS, 128), lambda k, j: (0, 0)),
        out_shape=jax.ShapeDtypeStruct((_INNER_STEPS, 128), jnp.float32),
        scratch_shapes=[
            pltpu.VMEM((NT, nl, TR), jnp.float32),
            pltpu.VMEM((128, 128), jnp.float32),
            pltpu.VMEM((8, 128), jnp.float32),
            pltpu.VMEM((nl, 128), jnp.float32),
            pltpu.VMEM((nl, 1), jnp.float32),
            pltpu.VMEM((nl, 128), jnp.float32),
            pltpu.VMEM((nl, 128), jnp.float32),
            pltpu.VMEM((1, 128), jnp.float32),
        ],
    )(lig_feat, rec_feat_pad, lig_cpad, rec_crd_pad, rc_fat, R0, t0)


_make_edge_agg_cached = functools.lru_cache(maxsize=None)(_make_edge_agg)


def kernel(rec_x, rec_coord, rec_edge_index, rec_edge_attr, lig_x, lig_coord,
           lig_edge_index, lig_edge_attr, Wrn, Wln, Wre, Wle, rec_Wproj,
           rec_We1, rec_We2, rec_Wself, rec_Wnbr, lig_Wproj, lig_We1, lig_We2,
           lig_Wself, lig_Wnbr):
    n_rec, n_lig = rec_x.shape[0], lig_x.shape[0]

    gate_r = _edge_gate(rec_edge_attr, Wre, rec_We1, rec_We2, 2000)
    e_lig = lig_edge_attr.shape[0]
    e_lig_pad = 16384
    lig_eattr_p = jnp.concatenate(
        [lig_edge_attr,
         jnp.zeros((e_lig_pad - e_lig, lig_edge_attr.shape[1]),
                   lig_edge_attr.dtype)])
    lig_ei_p = jnp.concatenate(
        [lig_edge_index,
         jnp.zeros((2, e_lig_pad - e_lig), lig_edge_index.dtype)], axis=1)
    gate_l = _edge_gate(lig_eattr_p, Wle, lig_We1, lig_We2, 2048)

    h_r = _node_embed(rec_x, Wrn, rec_Wproj, 1000)
    h_l = _node_embed(lig_x, Wln, lig_Wproj, 1000)

    src_r, dst_r = rec_edge_index[0], rec_edge_index[1]
    src_l, dst_l = lig_ei_p[0], lig_ei_p[1]

    rec_agg, rec_npad = _make_edge_agg_cached(10000, 320000, 80)
    lig_agg, lig_npad = _make_edge_agg_cached(1000, 16384, 64)

    for _ in range(_NUM_LAYERS):
        pa = rec_agg(h_r, gate_r, src_r, dst_r)
        h_r = _layer_update(h_r, pa[:n_rec], pa[rec_npad:rec_npad + n_rec],
                            rec_Wself, rec_Wnbr, 1000)
    for _ in range(_NUM_LAYERS):
        pa = lig_agg(h_l, gate_l, src_l, dst_l)
        h_l = _layer_update(h_l, pa[:n_lig], pa[lig_npad:lig_npad + n_lig],
                            lig_Wself, lig_Wnbr, 1000)

    lig_c = lig_coord - lig_coord.mean(axis=0)
    rec_c = rec_coord - rec_coord.mean(axis=0)
    kr = jax.random.key(7)
    rot, _ = jnp.linalg.qr(
        jax.random.normal(jax.random.fold_in(kr, 0), (3, 3), dtype=jnp.float32))
    trans = jax.random.normal(jax.random.fold_in(kr, 1), (3,),
                              dtype=jnp.float32) * _TRANS_DIST

    TR = 1024
    nrp = ((n_rec + TR - 1) // TR) * TR
    rec_feat_pad = jnp.concatenate(
        [h_r, jnp.zeros((nrp - n_rec, 128), jnp.float32)])
    rec_crd_pad = jnp.zeros((8, nrp), jnp.float32).at[:3, :n_rec].set(rec_c.T)
    rc_fat = jnp.zeros((128, nrp), jnp.float32).at[:3, :n_rec].set(
        rec_c.T).astype(jnp.bfloat16)
    lig_cpad = jnp.zeros((n_lig, 128), jnp.float32).at[:, :3].set(lig_c)
    R0 = jnp.zeros((128, 128), jnp.float32).at[:3, :3].set(rot.T)
    t0 = jnp.zeros((8, 128), jnp.float32).at[0, :3].set(trans)

    us = _dock(h_l, rec_feat_pad, lig_cpad, rec_crd_pad, rc_fat, R0, t0,
               n_lig, n_rec, TR)
    return us[:, 0]


# speedup vs baseline: 2.9179x; 1.1212x over previous
import functools

import jax
import jax.numpy as jnp
from jax import lax
from jax.experimental import pallas as pl
from jax.experimental.pallas import tpu as pltpu
from jax.experimental.pallas import tpu_sc as plsc

_NUM_LAYERS = 3
_INNER_STEPS = 4
_INNER_LR = 0.1
_TRANS_DIST = 1.0

_NC = 2
_NS = 16
_NW = _NC * _NS


def _make_edge_agg(n_nodes, n_edges, K):
    assert n_edges % (_NW * K) == 0 and K % 8 == 0 and K <= 128
    chunk = n_edges // _NW
    iters = chunk // K
    slab = ((n_nodes + _NS - 1) // _NS + 15) // 16 * 16
    n_pad = slab * _NS
    ZB = 40 if slab % 40 == 0 else (64 if slab % 64 == 0 else 8)
    assert slab % ZB == 0
    mesh = plsc.VectorSubcoreMesh(core_axis_name="c", subcore_axis_name="s",
                                  num_cores=_NC, num_subcores=_NS)

    assert iters % 2 == 0

    @functools.partial(
        pl.kernel,
        out_type=jax.ShapeDtypeStruct((_NC * n_pad, 128), jnp.float32),
        mesh=mesh,
        scratch_types=[
            [pltpu.VMEM((K,), jnp.int32) for _ in range(2)],
            [pltpu.VMEM((K,), jnp.int32) for _ in range(2)],
            [pltpu.VMEM((K, 128), jnp.float32) for _ in range(2)],
            [pltpu.VMEM((K, 128), jnp.float32) for _ in range(2)],
            pltpu.VMEM((ZB, 128), jnp.float32),
            pltpu.VMEM_SHARED((n_pad, 128), jnp.float32),
            [pltpu.SemaphoreType.DMA for _ in range(2)],
            [pltpu.SemaphoreType.DMA for _ in range(2)],
            [pltpu.SemaphoreType.DMA for _ in range(2)],
        ],
    )
    def body(h, gate, src, dst, out, src_v, dst_v, rows_v, gate_v, zbuf,
             shared, sem_g, sem_t, sem_s):
        c = lax.axis_index("c")
        s = lax.axis_index("s")

        def zfill(i, carry):
            for r in range(8):
                zbuf[i, pl.ds(r * 16, 16)] = jnp.zeros((16,), jnp.float32)
            return carry

        lax.fori_loop(0, ZB, zfill, 0)
        off = s * slab

        def zslab(i, carry):
            pltpu.sync_copy(zbuf, shared.at[pl.ds(off + i * ZB, ZB)])
            return carry

        lax.fori_loop(0, slab // ZB, zslab, 0)
        plsc.subcore_barrier()

        base = (c * _NS + s) * chunk

        def load(i, b):
            pltpu.sync_copy(src.at[pl.ds(base + i * K, K)], src_v[b])
            pltpu.sync_copy(dst.at[pl.ds(base + i * K, K)], dst_v[b])
            pltpu.async_copy(h.at[src_v[b]], rows_v[b], sem_g[b])
            pltpu.async_copy(gate.at[pl.ds(base + i * K, K)], gate_v[b], sem_t[b])

        def mul(b):
            def mrow(j, carry2):
                for r in range(8):
                    sl = pl.ds(r * 16, 16)
                    rows_v[b][j, sl] = rows_v[b][j, sl] * gate_v[b][j, sl]
                return carry2

            lax.fori_loop(0, K, mrow, 0)

        def drain(b):
            pltpu.make_async_copy(rows_v[b], shared.at[dst_v[b]], sem_s[b]).wait()

        load(0, 0)

        def pair(g, carry):
            i0 = 2 * g

            @pl.when(g > 0)
            def _():
                drain(1)

            load(i0 + 1, 1)
            pltpu.make_async_copy(h.at[src_v[0]], rows_v[0], sem_g[0]).wait()
            pltpu.make_async_copy(gate.at[pl.ds(0, K)], gate_v[0], sem_t[0]).wait()
            mul(0)
            pltpu.async_copy(rows_v[0], shared.at[dst_v[0]], sem_s[0], add=True)

            pltpu.make_async_copy(h.at[src_v[1]], rows_v[1], sem_g[1]).wait()
            pltpu.make_async_copy(gate.at[pl.ds(0, K)], gate_v[1], sem_t[1]).wait()
            mul(1)
            pltpu.async_copy(rows_v[1], shared.at[dst_v[1]], sem_s[1], add=True)
            drain(0)

            @pl.when(i0 + 2 < iters)
            def _():
                load(i0 + 2, 0)

            return carry

        lax.fori_loop(0, iters // 2, pair, 0)
        drain(1)
        plsc.subcore_barrier()
        pltpu.sync_copy(shared.at[pl.ds(off, slab)],
                        out.at[pl.ds(c * n_pad + off, slab)])

    return body, n_pad


def _node_embed(x, Wa, Wb, tile):
    n = x.shape[0]
    assert n % tile == 0

    def f(x_ref, wa_ref, wb_ref, o_ref):
        t = jnp.dot(x_ref[...].astype(jnp.bfloat16),
                    wa_ref[...].astype(jnp.bfloat16),
                    preferred_element_type=jnp.float32)
        o_ref[...] = jax.nn.relu(
            jnp.dot(t.astype(jnp.bfloat16), wb_ref[...].astype(jnp.bfloat16),
                    preferred_element_type=jnp.float32))

    return pl.pallas_call(
        f,
        grid=(n // tile,),
        in_specs=[
            pl.BlockSpec((tile, 128), lambda i: (i, 0)),
            pl.BlockSpec((128, 128), lambda i: (0, 0)),
            pl.BlockSpec((128, 128), lambda i: (0, 0)),
        ],
        out_specs=pl.BlockSpec((tile, 128), lambda i: (i, 0)),
        out_shape=jax.ShapeDtypeStruct((n, 128), jnp.float32),
    )(x, Wa, Wb)


def _edge_gate(eattr, We, We1, We2, tile):
    e, din = eattr.shape
    assert e % tile == 0
    deh = We1.shape[1]

    def f(e_ref, w_ref, w1_ref, w2_ref, o_ref):
        t = jnp.dot(e_ref[...].astype(jnp.bfloat16),
                    w_ref[...].astype(jnp.bfloat16),
                    preferred_element_type=jnp.float32)
        t = jnp.dot(t.astype(jnp.bfloat16), w1_ref[...].astype(jnp.bfloat16),
                    preferred_element_type=jnp.float32)
        o_ref[...] = jnp.dot(jax.nn.relu(t).astype(jnp.bfloat16),
                             w2_ref[...].astype(jnp.bfloat16),
                             preferred_element_type=jnp.float32)

    return pl.pallas_call(
        f,
        grid=(e // tile,),
        in_specs=[
            pl.BlockSpec((tile, din), lambda i: (i, 0)),
            pl.BlockSpec(We.shape, lambda i: (0, 0)),
            pl.BlockSpec(We1.shape, lambda i: (0, 0)),
            pl.BlockSpec(We2.shape, lambda i: (0, 0)),
        ],
        out_specs=pl.BlockSpec((tile, 128), lambda i: (i, 0)),
        out_shape=jax.ShapeDtypeStruct((e, 128), jnp.float32),
    )(eattr, We, We1, We2)


def _layer_update(h, a0, a1, Wself, Wnbr, tile):
    n = h.shape[0]
    assert n % tile == 0

    def f(h_ref, a0_ref, a1_ref, ws_ref, wn_ref, o_ref):
        agg = a0_ref[...] + a1_ref[...]
        o_ref[...] = jax.nn.relu(
            jnp.dot(h_ref[...].astype(jnp.bfloat16),
                    ws_ref[...].astype(jnp.bfloat16),
                    preferred_element_type=jnp.float32)
            + jnp.dot(agg.astype(jnp.bfloat16),
                      wn_ref[...].astype(jnp.bfloat16),
                      preferred_element_type=jnp.float32))

    return pl.pallas_call(
        f,
        grid=(n // tile,),
        in_specs=[
            pl.BlockSpec((tile, 128), lambda i: (i, 0)),
            pl.BlockSpec((tile, 128), lambda i: (i, 0)),
            pl.BlockSpec((tile, 128), lambda i: (i, 0)),
            pl.BlockSpec((128, 128), lambda i: (0, 0)),
            pl.BlockSpec((128, 128), lambda i: (0, 0)),
        ],
        out_specs=pl.BlockSpec((tile, 128), lambda i: (i, 0)),
        out_shape=jax.ShapeDtypeStruct((n, 128), jnp.float32),
    )(h, a0, a1, Wself, Wnbr)


def _dock(lig_feat, rec_feat_pad, lig_cpad, rec_crd_pad, rc_fat, R0, t0,
          nl, nr, TR):
    nrp = rec_feat_pad.shape[0]
    NT = nrp // TR
    c1 = -2.0 / (nl * nr)
    eps = 1e-12
    bf16 = jnp.bfloat16

    def f(lf_ref, rf_ref, lc_ref, rc_ref, rcf_ref, r0_ref, t0_ref, us_ref,
          atn_s, R_s, t_s, new_s, ln2_s, acc_s, wr_s, u_s):
        k = pl.program_id(0)
        j = pl.program_id(1)

        @pl.when(jnp.logical_and(k == 0, j == 0))
        def _init():
            R_s[...] = r0_ref[...]
            t_s[...] = t0_ref[...]

        @pl.when(k == 0)
        def _atn():
            atn_s[j] = lax.dot_general(
                lf_ref[...].astype(bf16), rf_ref[...].astype(bf16),
                (((1,), (1,)), ((), ())), preferred_element_type=jnp.float32)

        @pl.when(j == 0)
        def _step_init():
            new = jnp.dot(lc_ref[...].astype(bf16), R_s[...].astype(bf16),
                          preferred_element_type=jnp.float32) + t_s[0:1, :]
            new_s[...] = new
            nx = new[:, 0:1]
            ny = new[:, 1:2]
            nz = new[:, 2:3]
            ln2_s[...] = (nx * nx + nz * nz) + ny * ny
            acc_s[...] = jnp.zeros_like(acc_s)
            wr_s[...] = jnp.zeros_like(wr_s)
            u_s[...] = jnp.zeros_like(u_s)

        cross = lax.dot_general(new_s[...].astype(bf16), rcf_ref[...],
                                (((1,), (0,)), ((), ())),
                                preferred_element_type=jnp.float32)
        rx = rc_ref[0:1, :]
        ry = rc_ref[1:2, :]
        rz = rc_ref[2:3, :]
        rn2 = (rx * rx + rz * rz) + ry * ry
        d2 = ln2_s[...] + rn2 - 2.0 * cross
        m = jnp.maximum(d2, eps)
        dd = jnp.sqrt(m)
        dd2 = dd * dd
        atn = atn_s[j]
        q = atn / dd2
        u_s[...] += jnp.full((1, 128), jnp.sum(q), jnp.float32)
        w = q / dd2 * (d2 > eps).astype(jnp.float32)
        acc_s[:, 0:1] += jnp.sum(w, axis=1, keepdims=True)
        wr_s[:, 0:1] += jnp.sum(w * rx, axis=1, keepdims=True)
        wr_s[:, 1:2] += jnp.sum(w * ry, axis=1, keepdims=True)
        wr_s[:, 2:3] += jnp.sum(w * rz, axis=1, keepdims=True)

        @pl.when(j == NT - 1)
        def _finish():
            us_ref[pl.ds(k, 1), :] = u_s[...] * (1.0 / (nl * nr))
            G = c1 * (new_s[...] * acc_s[:, 0:1] - wr_s[...])
            gR = lax.dot_general(lc_ref[...].astype(bf16), G.astype(bf16),
                                 (((0,), (0,)), ((), ())),
                                 preferred_element_type=jnp.float32)
            gt = jnp.sum(G, axis=0, keepdims=True)
            R_s[...] = R_s[...] - _INNER_LR * gR
            t_s[0:1, :] = t_s[0:1, :] - _INNER_LR * gt

    return pl.pallas_call(
        f,
        grid=(_INNER_STEPS, NT),
        in_specs=[
            pl.BlockSpec((nl, 128), lambda k, j: (0, 0)),
            pl.BlockSpec((TR, 128), lambda k, j: (j, 0)),
            pl.BlockSpec((nl, 128), lambda k, j: (0, 0)),
            pl.BlockSpec((8, TR), lambda k, j: (0, j)),
            pl.BlockSpec((128, TR), lambda k, j: (0, j)),
            pl.BlockSpec((128, 128), lambda k, j: (0, 0)),
            pl.BlockSpec((8, 128), lambda k, j: (0, 0)),
        ],
        out_specs=pl.BlockSpec((_INNER_STEPS, 128), lambda k, j: (0, 0)),
        out_shape=jax.ShapeDtypeStruct((_INNER_STEPS, 128), jnp.float32),
        scratch_shapes=[
            pltpu.VMEM((NT, nl, TR), jnp.float32),
            pltpu.VMEM((128, 128), jnp.float32),
            pltpu.VMEM((8, 128), jnp.float32),
            pltpu.VMEM((nl, 128), jnp.float32),
            pltpu.VMEM((nl, 1), jnp.float32),
            pltpu.VMEM((nl, 128), jnp.float32),
            pltpu.VMEM((nl, 128), jnp.float32),
            pltpu.VMEM((1, 128), jnp.float32),
        ],
    )(lig_feat, rec_feat_pad, lig_cpad, rec_crd_pad, rc_fat, R0, t0)


_make_edge_agg_cached = functools.lru_cache(maxsize=None)(_make_edge_agg)


def kernel(rec_x, rec_coord, rec_edge_index, rec_edge_attr, lig_x, lig_coord,
           lig_edge_index, lig_edge_attr, Wrn, Wln, Wre, Wle, rec_Wproj,
           rec_We1, rec_We2, rec_Wself, rec_Wnbr, lig_Wproj, lig_We1, lig_We2,
           lig_Wself, lig_Wnbr):
    n_rec, n_lig = rec_x.shape[0], lig_x.shape[0]

    e_rec = rec_edge_attr.shape[0]
    e_rec_pad = 322560
    rec_eattr_p = jnp.concatenate(
        [rec_edge_attr,
         jnp.zeros((e_rec_pad - e_rec, rec_edge_attr.shape[1]),
                   rec_edge_attr.dtype)])
    rec_ei_p = jnp.concatenate(
        [rec_edge_index,
         jnp.zeros((2, e_rec_pad - e_rec), rec_edge_index.dtype)], axis=1)
    gate_r = _edge_gate(rec_eattr_p, Wre, rec_We1, rec_We2, 2016)
    e_lig = lig_edge_attr.shape[0]
    e_lig_pad = 16384
    lig_eattr_p = jnp.concatenate(
        [lig_edge_attr,
         jnp.zeros((e_lig_pad - e_lig, lig_edge_attr.shape[1]),
                   lig_edge_attr.dtype)])
    lig_ei_p = jnp.concatenate(
        [lig_edge_index,
         jnp.zeros((2, e_lig_pad - e_lig), lig_edge_index.dtype)], axis=1)
    gate_l = _edge_gate(lig_eattr_p, Wle, lig_We1, lig_We2, 2048)

    h_r = _node_embed(rec_x, Wrn, rec_Wproj, 1000)
    h_l = _node_embed(lig_x, Wln, lig_Wproj, 1000)

    src_r, dst_r = rec_ei_p[0], rec_ei_p[1]
    src_l, dst_l = lig_ei_p[0], lig_ei_p[1]

    rec_agg, rec_npad = _make_edge_agg_cached(10000, e_rec_pad, 80)
    lig_agg, lig_npad = _make_edge_agg_cached(1000, e_lig_pad, 128)

    for _ in range(_NUM_LAYERS):
        pa = rec_agg(h_r, gate_r, src_r, dst_r)
        h_r = _layer_update(h_r, pa[:n_rec], pa[rec_npad:rec_npad + n_rec],
                            rec_Wself, rec_Wnbr, 1000)
    for _ in range(_NUM_LAYERS):
        pa = lig_agg(h_l, gate_l, src_l, dst_l)
        h_l = _layer_update(h_l, pa[:n_lig], pa[lig_npad:lig_npad + n_lig],
                            lig_Wself, lig_Wnbr, 1000)

    lig_c = lig_coord - lig_coord.mean(axis=0)
    rec_c = rec_coord - rec_coord.mean(axis=0)
    kr = jax.random.key(7)
    rot, _ = jnp.linalg.qr(
        jax.random.normal(jax.random.fold_in(kr, 0), (3, 3), dtype=jnp.float32))
    trans = jax.random.normal(jax.random.fold_in(kr, 1), (3,),
                              dtype=jnp.float32) * _TRANS_DIST

    TR = 1024
    nrp = ((n_rec + TR - 1) // TR) * TR
    rec_feat_pad = jnp.concatenate(
        [h_r, jnp.zeros((nrp - n_rec, 128), jnp.float32)])
    rec_crd_pad = jnp.zeros((8, nrp), jnp.float32).at[:3, :n_rec].set(rec_c.T)
    rc_fat = jnp.zeros((128, nrp), jnp.float32).at[:3, :n_rec].set(
        rec_c.T).astype(jnp.bfloat16)
    lig_cpad = jnp.zeros((n_lig, 128), jnp.float32).at[:, :3].set(lig_c)
    R0 = jnp.zeros((128, 128), jnp.float32).at[:3, :3].set(rot.T)
    t0 = jnp.zeros((8, 128), jnp.float32).at[0, :3].set(trans)

    us = _dock(h_l, rec_feat_pad, lig_cpad, rec_crd_pad, rc_fat, R0, t0,
               n_lig, n_rec, TR)
    return us[:, 0]


# speedup vs baseline: 2.9864x; 1.0235x over previous
import functools

import jax
import jax.numpy as jnp
from jax import lax
from jax.experimental import pallas as pl
from jax.experimental.pallas import tpu as pltpu
from jax.experimental.pallas import tpu_sc as plsc

_NUM_LAYERS = 3
_INNER_STEPS = 4
_INNER_LR = 0.1
_TRANS_DIST = 1.0

_NC = 2
_NS = 16
_NW = _NC * _NS


def _make_edge_agg(n_nodes, n_edges, K):
    assert n_edges % (_NW * K) == 0 and K % 8 == 0 and K <= 128
    chunk = n_edges // _NW
    iters = chunk // K
    slab = ((n_nodes + _NS - 1) // _NS + 15) // 16 * 16
    n_pad = slab * _NS
    ZB = 40 if slab % 40 == 0 else (64 if slab % 64 == 0 else 8)
    assert slab % ZB == 0
    mesh = plsc.VectorSubcoreMesh(core_axis_name="c", subcore_axis_name="s",
                                  num_cores=_NC, num_subcores=_NS)

    assert iters % 2 == 0

    @functools.partial(
        pl.kernel,
        out_type=jax.ShapeDtypeStruct((_NC * n_pad, 128), jnp.float32),
        mesh=mesh,
        scratch_types=[
            [pltpu.VMEM((K,), jnp.int32) for _ in range(2)],
            [pltpu.VMEM((K,), jnp.int32) for _ in range(2)],
            [pltpu.VMEM((K, 128), jnp.float32) for _ in range(2)],
            [pltpu.VMEM((K, 128), jnp.float32) for _ in range(2)],
            pltpu.VMEM((ZB, 128), jnp.float32),
            pltpu.VMEM_SHARED((n_pad, 128), jnp.float32),
            [pltpu.SemaphoreType.DMA for _ in range(2)],
            [pltpu.SemaphoreType.DMA for _ in range(2)],
            [pltpu.SemaphoreType.DMA for _ in range(2)],
        ],
    )
    def body(h, gate, src, dst, out, src_v, dst_v, rows_v, gate_v, zbuf,
             shared, sem_g, sem_t, sem_s):
        c = lax.axis_index("c")
        s = lax.axis_index("s")

        def zfill(i, carry):
            for r in range(8):
                zbuf[i, pl.ds(r * 16, 16)] = jnp.zeros((16,), jnp.float32)
            return carry

        lax.fori_loop(0, ZB, zfill, 0)
        off = s * slab

        def zslab(i, carry):
            pltpu.sync_copy(zbuf, shared.at[pl.ds(off + i * ZB, ZB)])
            return carry

        lax.fori_loop(0, slab // ZB, zslab, 0)
        plsc.subcore_barrier()

        base = (c * _NS + s) * chunk

        def load(i, b):
            pltpu.sync_copy(src.at[pl.ds(base + i * K, K)], src_v[b])
            pltpu.sync_copy(dst.at[pl.ds(base + i * K, K)], dst_v[b])
            pltpu.async_copy(h.at[src_v[b]], rows_v[b], sem_g[b])
            pltpu.async_copy(gate.at[pl.ds(base + i * K, K)], gate_v[b], sem_t[b])

        def mul(b):
            def mrow(j2, carry2):
                for u in range(2):
                    j = 2 * j2 + u
                    for r in range(8):
                        sl = pl.ds(r * 16, 16)
                        rows_v[b][j, sl] = rows_v[b][j, sl] * gate_v[b][j, sl]
                return carry2

            lax.fori_loop(0, K // 2, mrow, 0)

        def drain(b):
            pltpu.make_async_copy(rows_v[b], shared.at[dst_v[b]], sem_s[b]).wait()

        load(0, 0)

        def pair(g, carry):
            i0 = 2 * g

            @pl.when(g > 0)
            def _():
                drain(1)

            load(i0 + 1, 1)
            pltpu.make_async_copy(h.at[src_v[0]], rows_v[0], sem_g[0]).wait()
            pltpu.make_async_copy(gate.at[pl.ds(0, K)], gate_v[0], sem_t[0]).wait()
            mul(0)
            pltpu.async_copy(rows_v[0], shared.at[dst_v[0]], sem_s[0], add=True)

            pltpu.make_async_copy(h.at[src_v[1]], rows_v[1], sem_g[1]).wait()
            pltpu.make_async_copy(gate.at[pl.ds(0, K)], gate_v[1], sem_t[1]).wait()
            mul(1)
            pltpu.async_copy(rows_v[1], shared.at[dst_v[1]], sem_s[1], add=True)
            drain(0)

            @pl.when(i0 + 2 < iters)
            def _():
                load(i0 + 2, 0)

            return carry

        lax.fori_loop(0, iters // 2, pair, 0)
        drain(1)
        plsc.subcore_barrier()
        pltpu.sync_copy(shared.at[pl.ds(off, slab)],
                        out.at[pl.ds(c * n_pad + off, slab)])

    return body, n_pad


def _node_embed(x, Wa, Wb, tile):
    n = x.shape[0]
    assert n % tile == 0

    def f(x_ref, wa_ref, wb_ref, o_ref):
        t = jnp.dot(x_ref[...].astype(jnp.bfloat16),
                    wa_ref[...].astype(jnp.bfloat16),
                    preferred_element_type=jnp.float32)
        o_ref[...] = jax.nn.relu(
            jnp.dot(t.astype(jnp.bfloat16), wb_ref[...].astype(jnp.bfloat16),
                    preferred_element_type=jnp.float32))

    return pl.pallas_call(
        f,
        grid=(n // tile,),
        in_specs=[
            pl.BlockSpec((tile, 128), lambda i: (i, 0)),
            pl.BlockSpec((128, 128), lambda i: (0, 0)),
            pl.BlockSpec((128, 128), lambda i: (0, 0)),
        ],
        out_specs=pl.BlockSpec((tile, 128), lambda i: (i, 0)),
        out_shape=jax.ShapeDtypeStruct((n, 128), jnp.float32),
    )(x, Wa, Wb)


def _edge_gate(eattr, We, We1, We2, tile):
    e, din = eattr.shape
    assert e % tile == 0
    deh = We1.shape[1]

    def f(e_ref, w_ref, w1_ref, w2_ref, o_ref):
        t = jnp.dot(e_ref[...].astype(jnp.bfloat16),
                    w_ref[...].astype(jnp.bfloat16),
                    preferred_element_type=jnp.float32)
        t = jnp.dot(t.astype(jnp.bfloat16), w1_ref[...].astype(jnp.bfloat16),
                    preferred_element_type=jnp.float32)
        o_ref[...] = jnp.dot(jax.nn.relu(t).astype(jnp.bfloat16),
                             w2_ref[...].astype(jnp.bfloat16),
                             preferred_element_type=jnp.float32)

    return pl.pallas_call(
        f,
        grid=(e // tile,),
        in_specs=[
            pl.BlockSpec((tile, din), lambda i: (i, 0)),
            pl.BlockSpec(We.shape, lambda i: (0, 0)),
            pl.BlockSpec(We1.shape, lambda i: (0, 0)),
            pl.BlockSpec(We2.shape, lambda i: (0, 0)),
        ],
        out_specs=pl.BlockSpec((tile, 128), lambda i: (i, 0)),
        out_shape=jax.ShapeDtypeStruct((e, 128), jnp.float32),
    )(eattr, We, We1, We2)


def _layer_update(h, a0, a1, Wself, Wnbr, tile):
    n = h.shape[0]
    assert n % tile == 0

    def f(h_ref, a0_ref, a1_ref, ws_ref, wn_ref, o_ref):
        agg = a0_ref[...] + a1_ref[...]
        o_ref[...] = jax.nn.relu(
            jnp.dot(h_ref[...].astype(jnp.bfloat16),
                    ws_ref[...].astype(jnp.bfloat16),
                    preferred_element_type=jnp.float32)
            + jnp.dot(agg.astype(jnp.bfloat16),
                      wn_ref[...].astype(jnp.bfloat16),
                      preferred_element_type=jnp.float32))

    return pl.pallas_call(
        f,
        grid=(n // tile,),
        in_specs=[
            pl.BlockSpec((tile, 128), lambda i: (i, 0)),
            pl.BlockSpec((tile, 128), lambda i: (i, 0)),
            pl.BlockSpec((tile, 128), lambda i: (i, 0)),
            pl.BlockSpec((128, 128), lambda i: (0, 0)),
            pl.BlockSpec((128, 128), lambda i: (0, 0)),
        ],
        out_specs=pl.BlockSpec((tile, 128), lambda i: (i, 0)),
        out_shape=jax.ShapeDtypeStruct((n, 128), jnp.float32),
    )(h, a0, a1, Wself, Wnbr)


def _dock(lig_feat, rec_feat_pad, lig_cpad, rec_crd_pad, rc_fat, R0, t0,
          nl, nr, TR):
    nrp = rec_feat_pad.shape[0]
    NT = nrp // TR
    c1 = -2.0 / (nl * nr)
    eps = 1e-12
    bf16 = jnp.bfloat16

    def f(lf_ref, rf_ref, lc_ref, rc_ref, rcf_ref, r0_ref, t0_ref, us_ref,
          atn_s, R_s, t_s, new_s, ln2_s, acc_s, wr_s, u_s):
        k = pl.program_id(0)
        j = pl.program_id(1)

        @pl.when(jnp.logical_and(k == 0, j == 0))
        def _init():
            R_s[...] = r0_ref[...]
            t_s[...] = t0_ref[...]

        @pl.when(k == 0)
        def _atn():
            atn_s[j] = lax.dot_general(
                lf_ref[...].astype(bf16), rf_ref[...].astype(bf16),
                (((1,), (1,)), ((), ())), preferred_element_type=jnp.float32)

        @pl.when(j == 0)
        def _step_init():
            new = jnp.dot(lc_ref[...].astype(bf16), R_s[...].astype(bf16),
                          preferred_element_type=jnp.float32) + t_s[0:1, :]
            new_s[...] = new
            nx = new[:, 0:1]
            ny = new[:, 1:2]
            nz = new[:, 2:3]
            ln2_s[...] = (nx * nx + nz * nz) + ny * ny
            acc_s[...] = jnp.zeros_like(acc_s)
            wr_s[...] = jnp.zeros_like(wr_s)
            u_s[...] = jnp.zeros_like(u_s)

        cross = lax.dot_general(new_s[...].astype(bf16), rcf_ref[...],
                                (((1,), (0,)), ((), ())),
                                preferred_element_type=jnp.float32)
        rx = rc_ref[0:1, :]
        ry = rc_ref[1:2, :]
        rz = rc_ref[2:3, :]
        rn2 = (rx * rx + rz * rz) + ry * ry
        d2 = ln2_s[...] + rn2 - 2.0 * cross
        m = jnp.maximum(d2, eps)
        atn = atn_s[j]
        q = atn / m
        u_s[...] += jnp.full((1, 128), jnp.sum(q), jnp.float32)
        w = q / m * (d2 > eps).astype(jnp.float32)
        acc_s[:, 0:1] += jnp.sum(w, axis=1, keepdims=True)
        wr_s[:, 0:1] += jnp.sum(w * rx, axis=1, keepdims=True)
        wr_s[:, 1:2] += jnp.sum(w * ry, axis=1, keepdims=True)
        wr_s[:, 2:3] += jnp.sum(w * rz, axis=1, keepdims=True)

        @pl.when(j == NT - 1)
        def _finish():
            us_ref[pl.ds(k, 1), :] = u_s[...] * (1.0 / (nl * nr))
            G = c1 * (new_s[...] * acc_s[:, 0:1] - wr_s[...])
            gR = lax.dot_general(lc_ref[...].astype(bf16), G.astype(bf16),
                                 (((0,), (0,)), ((), ())),
                                 preferred_element_type=jnp.float32)
            gt = jnp.sum(G, axis=0, keepdims=True)
            R_s[...] = R_s[...] - _INNER_LR * gR
            t_s[0:1, :] = t_s[0:1, :] - _INNER_LR * gt

    return pl.pallas_call(
        f,
        grid=(_INNER_STEPS, NT),
        in_specs=[
            pl.BlockSpec((nl, 128), lambda k, j: (0, 0)),
            pl.BlockSpec((TR, 128), lambda k, j: (j, 0)),
            pl.BlockSpec((nl, 128), lambda k, j: (0, 0)),
            pl.BlockSpec((8, TR), lambda k, j: (0, j)),
            pl.BlockSpec((128, TR), lambda k, j: (0, j)),
            pl.BlockSpec((128, 128), lambda k, j: (0, 0)),
            pl.BlockSpec((8, 128), lambda k, j: (0, 0)),
        ],
        out_specs=pl.BlockSpec((_INNER_STEPS, 128), lambda k, j: (0, 0)),
        out_shape=jax.ShapeDtypeStruct((_INNER_STEPS, 128), jnp.float32),
        scratch_shapes=[
            pltpu.VMEM((NT, nl, TR), jnp.float32),
            pltpu.VMEM((128, 128), jnp.float32),
            pltpu.VMEM((8, 128), jnp.float32),
            pltpu.VMEM((nl, 128), jnp.float32),
            pltpu.VMEM((nl, 1), jnp.float32),
            pltpu.VMEM((nl, 128), jnp.float32),
            pltpu.VMEM((nl, 128), jnp.float32),
            pltpu.VMEM((1, 128), jnp.float32),
        ],
    )(lig_feat, rec_feat_pad, lig_cpad, rec_crd_pad, rc_fat, R0, t0)


_make_edge_agg_cached = functools.lru_cache(maxsize=None)(_make_edge_agg)


def kernel(rec_x, rec_coord, rec_edge_index, rec_edge_attr, lig_x, lig_coord,
           lig_edge_index, lig_edge_attr, Wrn, Wln, Wre, Wle, rec_Wproj,
           rec_We1, rec_We2, rec_Wself, rec_Wnbr, lig_Wproj, lig_We1, lig_We2,
           lig_Wself, lig_Wnbr):
    n_rec, n_lig = rec_x.shape[0], lig_x.shape[0]

    e_rec = rec_edge_attr.shape[0]
    e_rec_pad = 322560
    rec_eattr_p = jnp.concatenate(
        [rec_edge_attr,
         jnp.zeros((e_rec_pad - e_rec, rec_edge_attr.shape[1]),
                   rec_edge_attr.dtype)])
    rec_ei_p = jnp.concatenate(
        [rec_edge_index,
         jnp.zeros((2, e_rec_pad - e_rec), rec_edge_index.dtype)], axis=1)
    gate_r = _edge_gate(rec_eattr_p, Wre, rec_We1, rec_We2, 2016)
    e_lig = lig_edge_attr.shape[0]
    e_lig_pad = 16384
    lig_eattr_p = jnp.concatenate(
        [lig_edge_attr,
         jnp.zeros((e_lig_pad - e_lig, lig_edge_attr.shape[1]),
                   lig_edge_attr.dtype)])
    lig_ei_p = jnp.concatenate(
        [lig_edge_index,
         jnp.zeros((2, e_lig_pad - e_lig), lig_edge_index.dtype)], axis=1)
    gate_l = _edge_gate(lig_eattr_p, Wle, lig_We1, lig_We2, 2048)

    h_r = _node_embed(rec_x, Wrn, rec_Wproj, 1000)
    h_l = _node_embed(lig_x, Wln, lig_Wproj, 1000)

    src_r, dst_r = rec_ei_p[0], rec_ei_p[1]
    src_l, dst_l = lig_ei_p[0], lig_ei_p[1]

    rec_agg, rec_npad = _make_edge_agg_cached(10000, e_rec_pad, 80)
    lig_agg, lig_npad = _make_edge_agg_cached(1000, e_lig_pad, 128)

    for _ in range(_NUM_LAYERS):
        pa = rec_agg(h_r, gate_r, src_r, dst_r)
        h_r = _layer_update(h_r, pa[:n_rec], pa[rec_npad:rec_npad + n_rec],
                            rec_Wself, rec_Wnbr, 1000)
    for _ in range(_NUM_LAYERS):
        pa = lig_agg(h_l, gate_l, src_l, dst_l)
        h_l = _layer_update(h_l, pa[:n_lig], pa[lig_npad:lig_npad + n_lig],
                            lig_Wself, lig_Wnbr, 1000)

    lig_c = lig_coord - lig_coord.mean(axis=0)
    rec_c = rec_coord - rec_coord.mean(axis=0)
    kr = jax.random.key(7)
    rot, _ = jnp.linalg.qr(
        jax.random.normal(jax.random.fold_in(kr, 0), (3, 3), dtype=jnp.float32))
    trans = jax.random.normal(jax.random.fold_in(kr, 1), (3,),
                              dtype=jnp.float32) * _TRANS_DIST

    TR = 1024
    nrp = ((n_rec + TR - 1) // TR) * TR
    rec_feat_pad = jnp.concatenate(
        [h_r, jnp.zeros((nrp - n_rec, 128), jnp.float32)])
    rec_crd_pad = jnp.zeros((8, nrp), jnp.float32).at[:3, :n_rec].set(rec_c.T)
    rc_fat = jnp.zeros((128, nrp), jnp.float32).at[:3, :n_rec].set(
        rec_c.T).astype(jnp.bfloat16)
    lig_cpad = jnp.zeros((n_lig, 128), jnp.float32).at[:, :3].set(lig_c)
    R0 = jnp.zeros((128, 128), jnp.float32).at[:3, :3].set(rot.T)
    t0 = jnp.zeros((8, 128), jnp.float32).at[0, :3].set(trans)

    us = _dock(h_l, rec_feat_pad, lig_cpad, rec_crd_pad, rc_fat, R0, t0,
               n_lig, n_rec, TR)
    return us[:, 0]
